# baseline (speedup 1.0000x reference)
import functools

import jax
import jax.numpy as jnp
from jax import lax
from jax.experimental import pallas as pl
from jax.experimental.pallas import tpu as pltpu
from jax.experimental.pallas import tpu_sc as plsc

_BATCH = 4
_SEQ = 8192
_DIM = 1024
_ROWS_PER_CORE = _SEQ // 2
_CHUNK = 512
_NCHUNKS = _ROWS_PER_CORE // _CHUNK


def _broadcast_table(pos_embedding):
    mesh = plsc.ScalarSubcoreMesh(axis_name="c", num_cores=2)

    @functools.partial(
        pl.kernel,
        mesh=mesh,
        out_type=jax.ShapeDtypeStruct((_BATCH, _SEQ, _DIM), jnp.float32),
        scratch_types=[
            pltpu.VMEM_SHARED((_CHUNK, _DIM), jnp.float32),
            pltpu.VMEM_SHARED((_CHUNK, _DIM), jnp.float32),
            pltpu.SemaphoreType.DMA,
            pltpu.SemaphoreType.DMA,
            pltpu.SemaphoreType.DMA,
            pltpu.SemaphoreType.DMA,
        ],
    )
    def k(table_hbm, out_hbm, b0, b1, r0, r1, w0, w1):
        base = lax.axis_index("c") * _ROWS_PER_CORE
        bufs = (b0, b1)
        rsems = (r0, r1)
        wsems = (w0, w1)

        def start_read(i):
            return pltpu.async_copy(
                table_hbm.at[pl.ds(base + i * _CHUNK, _CHUNK)],
                bufs[i % 2], rsems[i % 2])

        reads = [None] * _NCHUNKS
        writes = [[] for _ in range(_NCHUNKS)]
        reads[0] = start_read(0)
        for i in range(_NCHUNKS):
            p = i % 2
            reads[i].wait()
            if i >= 1:
                for w in writes[i - 1]:
                    w.wait()
            if i + 1 < _NCHUNKS:
                reads[i + 1] = start_read(i + 1)
            row0 = base + i * _CHUNK
            for b in range(_BATCH):
                writes[i].append(pltpu.async_copy(
                    bufs[p], out_hbm.at[b, pl.ds(row0, _CHUNK)], wsems[p]))
        for w in writes[_NCHUNKS - 1]:
            w.wait()

    return k(pos_embedding)


def kernel(input_ids, pos_embedding):
    del input_ids
    return _broadcast_table(pos_embedding)


# speedup vs baseline: 1.3745x; 1.3745x over previous
import jax
import jax.numpy as jnp
from jax import lax
from jax.experimental import pallas as pl
from jax.experimental.pallas import tpu as pltpu
from jax.experimental.pallas import tpu_sc as plsc
from jax._src.pallas import core as pallas_core
from jax._src.pallas import mpmd

_BATCH = 4
_SEQ = 8192
_DIM = 1024

_SCS_ROWS = 2560
_SCS_PER_CORE = _SCS_ROWS // 2
_SCS_CHUNKS = (512, 512, 256)

_TEC_ROWS = _SEQ - _SCS_ROWS
_TEC_PER_WORKER = _TEC_ROWS // 32
_TEC_CHUNKS = (64, 64, 48)


def _broadcast_table(pos_embedding):
    vmesh = plsc.VectorSubcoreMesh(core_axis_name="c", subcore_axis_name="s")
    smesh = plsc.ScalarSubcoreMesh(axis_name="c", num_cores=2)

    def scs_fn(table_hbm, out_hbm, sbuf, tbuf):
        del tbuf
        base = lax.axis_index("c") * _SCS_PER_CORE
        off = 0
        for n in _SCS_CHUNKS:
            row0 = base + off
            pltpu.sync_copy(table_hbm.at[pl.ds(row0, n)], sbuf.at[pl.ds(0, n)])
            for b in range(_BATCH):
                pltpu.sync_copy(
                    sbuf.at[pl.ds(0, n)], out_hbm.at[b, pl.ds(row0, n)])
            off += n

    def tec_fn(table_hbm, out_hbm, sbuf, tbuf):
        del sbuf
        wid = lax.axis_index("s") * 2 + lax.axis_index("c")
        base = _SCS_ROWS + wid * _TEC_PER_WORKER
        off = 0
        for n in _TEC_CHUNKS:
            row0 = base + off
            pltpu.sync_copy(table_hbm.at[pl.ds(row0, n)], tbuf.at[pl.ds(0, n)])
            for b in range(_BATCH):
                pltpu.sync_copy(
                    tbuf.at[pl.ds(0, n)], out_hbm.at[b, pl.ds(row0, n)])
            off += n

    f = mpmd.mpmd_map(
        [(smesh, scs_fn), (vmesh, tec_fn)],
        out_types=jax.ShapeDtypeStruct((_BATCH, _SEQ, _DIM), jnp.float32),
        scratch_types=[
            pltpu.VMEM_SHARED((512, _DIM), jnp.float32),
            pallas_core.CoreMemorySpace(pltpu.VMEM, vmesh)(
                (64, _DIM), jnp.float32),
        ],
    )
    return f(pos_embedding)


def kernel(input_ids, pos_embedding):
    del input_ids
    return _broadcast_table(pos_embedding)
